# baseline (speedup 1.0000x reference)
import functools

import jax
import jax.numpy as jnp
from jax import lax
from jax.experimental import pallas as pl
from jax.experimental.pallas import tpu as pltpu
from jax.experimental.pallas import tpu_sc as plsc

N_NODES = 100000
D = 128
G = 1024

NC = 2
NS = 16
NW = NC * NS
R = 128
NB = 782
TCB = 160
SCNB = NB - TCB
BPT = -(-SCNB // NW)
ACC_ROWS = 1152
ZROWS = ACC_ROWS // NS
LAST_START = N_NODES - R


def _sc_partials(node_feat, idx_blocks):
    mesh = plsc.VectorSubcoreMesh(core_axis_name="c", subcore_axis_name="s")

    @functools.partial(
        pl.kernel,
        out_type=jax.ShapeDtypeStruct((NC, G, D), jnp.float32),
        mesh=mesh,
        scratch_types=[
            pltpu.VMEM((4, R, D), jnp.float32),
            pltpu.VMEM((BPT, R), jnp.int32),
            pltpu.VMEM((ZROWS, D), jnp.float32),
            pltpu.VMEM_SHARED((ACC_ROWS, D), jnp.float32),
            pltpu.SemaphoreType.DMA((4,)),
            pltpu.SemaphoreType.DMA((4,)),
            pltpu.SemaphoreType.DMA,
        ],
    )
    def body(feat_hbm, idx_hbm, out_hbm, rowbuf, idxv, zbuf, acc,
             lsem, ssem, isem):
        cid = lax.axis_index("c")
        sid = lax.axis_index("s")
        wid = cid * NS + sid

        idx_cp = pltpu.async_copy(idx_hbm.at[wid], idxv, isem)

        def load_desc(b):
            p = lax.rem(b, 4)
            rs = jnp.minimum((TCB + b * NW + wid) * R, LAST_START)
            return pltpu.make_async_copy(
                feat_hbm.at[pl.ds(rs, R)], rowbuf.at[p], lsem.at[p])

        def scat_desc(b):
            p = lax.rem(b, 4)
            return pltpu.make_async_copy(rowbuf.at[p], acc.at[idxv.at[b]],
                                         ssem.at[p])

        def start_load(b):
            @pl.when(b * NW + wid < SCNB)
            def _():
                load_desc(b).start()

        def wait_scatter(b):
            @pl.when(b * NW + wid < SCNB)
            def _():
                scat_desc(b).wait()

        start_load(0)
        start_load(1)

        def zrow(i, _):
            def zcol(j, _):
                zbuf[i, pl.ds(j * 16, 16)] = jnp.zeros((16,), jnp.float32)
                return 0
            return lax.fori_loop(0, D // 16, zcol, 0)
        lax.fori_loop(0, ZROWS, zrow, 0)
        pltpu.sync_copy(zbuf, acc.at[pl.ds(sid * ZROWS, ZROWS)])
        plsc.subcore_barrier()
        idx_cp.wait()

        def block(b, _):
            @pl.when(b >= 2)
            def _():
                wait_scatter(b - 2)
            start_load(b + 2)

            @pl.when(b * NW + wid < SCNB)
            def _():
                p = lax.rem(b, 4)
                load_desc(b).wait()
                pltpu.async_copy(rowbuf.at[p], acc.at[idxv.at[b]],
                                 ssem.at[p], add=True)
            return 0
        lax.fori_loop(0, BPT, block, 0)
        wait_scatter(BPT - 2)
        wait_scatter(BPT - 1)
        plsc.subcore_barrier()

        rows = G // NS
        pltpu.sync_copy(
            acc.at[pl.ds(sid * rows, rows)],
            out_hbm.at[cid].at[pl.ds(sid * rows, rows)],
        )

    return body(node_feat, idx_blocks)


def _tc_partial(node_feat, tc_ids):
    def body(ids_ref, rows_ref, o_ref):
        @pl.when(pl.program_id(0) == 0)
        def _():
            o_ref[...] = jnp.zeros_like(o_ref)

        ids = ids_ref[0, 0, :]
        onehot = (ids[:, None] ==
                  lax.broadcasted_iota(jnp.int32, (R, G), 1)).astype(
                      jnp.float32)
        o_ref[...] += lax.dot_general(
            onehot, rows_ref[...], (((0,), (0,)), ((), ())),
            preferred_element_type=jnp.float32)

    return pl.pallas_call(
        body,
        grid=(TCB,),
        in_specs=[
            pl.BlockSpec((1, 1, R), lambda b: (b, 0, 0)),
            pl.BlockSpec((R, D), lambda b: (b, 0)),
        ],
        out_specs=pl.BlockSpec((G, D), lambda b: (0, 0)),
        out_shape=jax.ShapeDtypeStruct((G, D), jnp.float32),
    )(tc_ids, node_feat)


def _sum_partials(sc_parts, tc_part):
    def add_body(p_ref, t_ref, o_ref):
        o_ref[...] = p_ref[0] + p_ref[1] + t_ref[...]

    return pl.pallas_call(
        add_body,
        out_shape=jax.ShapeDtypeStruct((G, D), jnp.float32),
    )(sc_parts, tc_part)


@jax.jit
def kernel(node_feat, batch):
    bid = batch.astype(jnp.int32)
    main = bid[: (NB - 1) * R].reshape(NB - 1, R)
    tail = jnp.concatenate(
        [jnp.full((R - (N_NODES - (NB - 1) * R),), G, jnp.int32),
         bid[(NB - 1) * R:]]
    ).reshape(1, R)
    pad = jnp.full((TCB + NW * BPT - NB, R), G, jnp.int32)
    blocks = jnp.concatenate([main, tail, pad], axis=0)

    tc_ids = blocks[:TCB].reshape(TCB, 1, R)
    sc_idx = (
        blocks[TCB:]
        .reshape(BPT, NW, R)
        .transpose(1, 0, 2)
    )

    sc_parts = _sc_partials(node_feat, sc_idx)
    tc_part = _tc_partial(node_feat, tc_ids)
    return _sum_partials(sc_parts, tc_part)


# speedup vs baseline: 2.5296x; 2.5296x over previous
import functools

import jax
import jax.numpy as jnp
from jax import lax
from jax.experimental import pallas as pl
from jax.experimental.pallas import tpu as pltpu
from jax.experimental.pallas import tpu_sc as plsc

N_NODES = 100000
D = 128
G = 1024

NC = 2
NS = 16
NW = NC * NS
R = 128
NB = 782
BPT = 25
NBUF = 6
ACC_ROWS = 1152
ZROWS = ACC_ROWS // NS
LAST_START = N_NODES - R


def _sc_partials(node_feat, idx_blocks):
    mesh = plsc.VectorSubcoreMesh(core_axis_name="c", subcore_axis_name="s")

    @functools.partial(
        pl.kernel,
        out_type=jax.ShapeDtypeStruct((NC, G, D), jnp.float32),
        mesh=mesh,
        scratch_types=[
            pltpu.VMEM((NBUF, R, D), jnp.float32),
            pltpu.VMEM((BPT, R), jnp.int32),
            pltpu.VMEM((ZROWS, D), jnp.float32),
            pltpu.VMEM_SHARED((ACC_ROWS, D), jnp.float32),
            pltpu.SemaphoreType.DMA((NBUF,)),
            pltpu.SemaphoreType.DMA((NBUF,)),
            pltpu.SemaphoreType.DMA,
        ],
    )
    def body(feat_hbm, idx_hbm, out_hbm, rowbuf, idxv, zbuf, acc,
             lsem, ssem, isem):
        cid = lax.axis_index("c")
        sid = lax.axis_index("s")
        wid = cid * NS + sid

        idx_cp = pltpu.async_copy(idx_hbm.at[wid], idxv, isem)

        def load_desc(b):
            p = lax.rem(b, NBUF)
            rs = jnp.minimum((b * NW + wid) * R, LAST_START)
            return pltpu.make_async_copy(
                feat_hbm.at[pl.ds(rs, R)], rowbuf.at[p], lsem.at[p])

        def scat_desc(b):
            p = lax.rem(b, NBUF)
            return pltpu.make_async_copy(rowbuf.at[p], acc.at[idxv.at[b]],
                                         ssem.at[p])

        def start_load(b):
            @pl.when(b * NW + wid < NB)
            def _():
                load_desc(b).start()

        def wait_scatter(b):
            @pl.when(b * NW + wid < NB)
            def _():
                scat_desc(b).wait()

        start_load(0)
        start_load(1)
        start_load(2)

        def zrow(i, _):
            def zcol(j, _):
                zbuf[i, pl.ds(j * 16, 16)] = jnp.zeros((16,), jnp.float32)
                return 0
            return lax.fori_loop(0, D // 16, zcol, 0)
        lax.fori_loop(0, ZROWS, zrow, 0)
        pltpu.sync_copy(zbuf, acc.at[pl.ds(sid * ZROWS, ZROWS)])
        plsc.subcore_barrier()
        idx_cp.wait()

        def block(b, _):
            @pl.when(b >= 3)
            def _():
                wait_scatter(b - 3)
            start_load(b + 3)

            @pl.when(b * NW + wid < NB)
            def _():
                p = lax.rem(b, NBUF)
                load_desc(b).wait()
                pltpu.async_copy(rowbuf.at[p], acc.at[idxv.at[b]],
                                 ssem.at[p], add=True)
            return 0
        lax.fori_loop(0, BPT, block, 0)
        wait_scatter(BPT - 3)
        wait_scatter(BPT - 2)
        wait_scatter(BPT - 1)
        plsc.subcore_barrier()

        rows = G // NS
        pltpu.sync_copy(
            acc.at[pl.ds(sid * rows, rows)],
            out_hbm.at[cid].at[pl.ds(sid * rows, rows)],
        )

    return body(node_feat, idx_blocks)


def _sum_partials(partials):
    def add_body(p_ref, o_ref):
        o_ref[...] = p_ref[0] + p_ref[1]

    return pl.pallas_call(
        add_body,
        out_shape=jax.ShapeDtypeStruct((G, D), jnp.float32),
    )(partials)


@jax.jit
def kernel(node_feat, batch):
    bid = batch.astype(jnp.int32)
    main = bid[: (NB - 1) * R].reshape(NB - 1, R)
    tail = jnp.concatenate(
        [jnp.full((R - (N_NODES - (NB - 1) * R),), G, jnp.int32),
         bid[(NB - 1) * R:]]
    ).reshape(1, R)
    pad = jnp.full((NW * BPT - NB, R), G, jnp.int32)
    idx_blocks = (
        jnp.concatenate([main, tail, pad], axis=0)
        .reshape(BPT, NW, R)
        .transpose(1, 0, 2)
    )

    partials = _sc_partials(node_feat, idx_blocks)
    return _sum_partials(partials)
